# Initial kernel scaffold; baseline (speedup 1.0000x reference)
import jax
import jax.numpy as jnp
from jax.experimental import pallas as pl


def kernel(x, edge_index, edge_attr, batch, node_emb, edge_emb, edge_enc_W, edge_enc_b, pre_W, pre_b, post_W, post_b, lin_W, lin_b, bn_gamma, bn_beta, mlp_W1, mlp_b1, mlp_W2, mlp_b2, mlp_W3, mlp_b3):
    raise NotImplementedError("write your pallas kernel here")


# speedup vs baseline: 13.3266x; 13.3266x over previous
import functools

from jax import lax as _lax

import jax
import jax.numpy as jnp
from jax import lax
from jax.experimental import pallas as pl
from jax.experimental.pallas import tpu as pltpu

N = 10000
E = 160000
G = 256
L = 4
T = 5
FIN = 75
FOUT = 15
FP = 128
BLK = 1000


def _pool_kernel(batch_ref, h_ref, out_ref, acc):
    i = pl.program_id(0)
    nblk = pl.num_programs(0)
    b = batch_ref[0]
    hb = h_ref[...]
    gids = lax.broadcasted_iota(jnp.int32, (G, BLK), 0)
    onehot = (gids == b).astype(jnp.float32)
    contrib = jnp.dot(onehot, hb, preferred_element_type=jnp.float32,
                      precision=lax.Precision.HIGHEST)

    @pl.when(i == 0)
    def _():
        acc[...] = jnp.zeros_like(acc)

    acc[...] += contrib

    @pl.when(i == nblk - 1)
    def _():
        out_ref[...] = acc[...]


def _mlp_kernel(g_ref, w1_ref, b1_ref, w2_ref, b2_ref, w3_ref, b3_ref, out_ref):
    def mm(a, b):
        return jnp.dot(a.astype(jnp.bfloat16), b.astype(jnp.bfloat16),
                       preferred_element_type=jnp.float32)
    g = jnp.maximum(mm(g_ref[...], w1_ref[...]) + b1_ref[...], 0.0)
    g = jnp.maximum(mm(g, w2_ref[...]) + b2_ref[...], 0.0)
    out_ref[...] = mm(g, w3_ref[...]) + b3_ref[...]


def _pad2(a, r, c):
    return jnp.pad(a, ((0, r - a.shape[0]), (0, c - a.shape[1])))


def _bf(a):
    return a.astype(jnp.bfloat16)


def _mm(a, b):
    return jnp.dot(_bf(a), _bf(b), preferred_element_type=jnp.float32)


def _es(spec, a, b):
    return jnp.einsum(spec, _bf(a), _bf(b), preferred_element_type=jnp.float32)


def kernel(*args):
    return _impl(*args)


def _impl(x, edge_index, edge_attr, batch, node_emb, edge_emb, edge_enc_W, edge_enc_b, pre_W, pre_b, post_W, post_b, lin_W, lin_b, bn_gamma, bn_beta, mlp_W1, mlp_b1, mlp_W2, mlp_b2, mlp_W3, mlp_b3):
    h = node_emb[x]
    src, dst = edge_index[0], edge_index[1]
    cnt = jax.ops.segment_sum(jnp.ones((E,), jnp.float32), dst, num_segments=N)
    cnt_c = jnp.maximum(cnt, 1.0)[:, None]
    empty = (cnt == 0.0)[:, None]
    TF = T * FIN
    for l in range(L):
        ee4 = _mm(edge_emb, edge_enc_W[l]) + edge_enc_b[l]
        A = (_es('nf,tfo->nto', h, pre_W[l][:, :FIN]) + pre_b[l]).reshape(N, TF)
        B = _es('nf,tfo->nto', h, pre_W[l][:, FIN:2 * FIN]).reshape(N, TF)
        C = _es('kf,tfo->kto', ee4, pre_W[l][:, 2 * FIN:]).reshape(4, TF)
        val = B[src] + C[edge_attr]
        S = jax.ops.segment_sum(val, dst, num_segments=N)
        Q = jax.ops.segment_sum(val * val, dst, num_segments=N)
        MN = jax.ops.segment_min(val, dst, num_segments=N)
        MX = jax.ops.segment_max(val, dst, num_segments=N)
        meanv = S / cnt_c
        mean = jnp.where(empty, 0.0, A + meanv)
        std = jnp.sqrt(jax.nn.relu(Q / cnt_c - meanv * meanv) + 1e-5)
        mn = jnp.where(empty, 0.0, A + MN)
        mx = jnp.where(empty, 0.0, A + MX)
        P = post_W[l]
        out = (_es('nf,tfo->nto', h, P[:, :FIN])
               + _es('ntf,tfo->nto', mean.reshape(N, T, FIN), P[:, FIN:2 * FIN])
               + _es('ntf,tfo->nto', mn.reshape(N, T, FIN), P[:, 2 * FIN:3 * FIN])
               + _es('ntf,tfo->nto', mx.reshape(N, T, FIN), P[:, 3 * FIN:4 * FIN])
               + _es('ntf,tfo->nto', std.reshape(N, T, FIN), P[:, 4 * FIN:])
               + post_b[l])
        out = _mm(out.reshape(N, T * FOUT), lin_W[l]) + lin_b[l]
        mu = out.mean(axis=0)
        var = out.var(axis=0)
        out = (out - mu) / jnp.sqrt(var + 1e-5) * bn_gamma[l] + bn_beta[l]
        h = jax.nn.relu(out)

    hp = _pad2(h, N, FP)
    g = pl.pallas_call(
        _pool_kernel,
        grid=(N // BLK,),
        in_specs=[pl.BlockSpec((1, 1, BLK), lambda i: (i, 0, 0)),
                  pl.BlockSpec((BLK, FP), lambda i: (i, 0))],
        out_specs=pl.BlockSpec((G, FP), lambda i: (0, 0)),
        out_shape=jax.ShapeDtypeStruct((G, FP), jnp.float32),
        scratch_shapes=[pltpu.VMEM((G, FP), jnp.float32)],
    )(batch.astype(jnp.int32).reshape(N // BLK, 1, BLK), hp)

    w1 = _pad2(mlp_W1, FP, FP)
    b1 = _pad2(mlp_b1.reshape(1, -1), 1, FP)
    w2 = _pad2(mlp_W2, FP, FP)
    b2 = _pad2(mlp_b2.reshape(1, -1), 1, FP)
    w3 = _pad2(mlp_W3, FP, FP)
    b3 = _pad2(mlp_b3.reshape(1, -1), 1, FP)
    out = pl.pallas_call(
        _mlp_kernel,
        out_shape=jax.ShapeDtypeStruct((G, FP), jnp.float32),
    )(g, w1, b1, w2, b2, w3, b3)
    return out[:, :1]


# speedup vs baseline: 33.6298x; 2.5235x over previous
import functools

import jax
import jax.numpy as jnp
from jax import lax
from jax.experimental import pallas as pl
from jax.experimental.pallas import tpu as pltpu
from jax.experimental.pallas import tpu_sc as plsc

N = 10000
E = 160000
G = 256
L = 4
T = 5
FIN = 75
FOUT = 15
FP = 128
BLK = 1000


def _pool_kernel(batch_ref, h_ref, out_ref, acc):
    i = pl.program_id(0)
    nblk = pl.num_programs(0)
    b = batch_ref[0]
    hb = h_ref[...]
    gids = lax.broadcasted_iota(jnp.int32, (G, BLK), 0)
    onehot = (gids == b).astype(jnp.float32)
    contrib = jnp.dot(onehot, hb, preferred_element_type=jnp.float32,
                      precision=lax.Precision.HIGHEST)

    @pl.when(i == 0)
    def _():
        acc[...] = jnp.zeros_like(acc)

    acc[...] += contrib

    @pl.when(i == nblk - 1)
    def _():
        out_ref[...] = acc[...]


def _mlp_kernel(g_ref, w1_ref, b1_ref, w2_ref, b2_ref, w3_ref, b3_ref, out_ref):
    def mm(a, b):
        return jnp.dot(a.astype(jnp.bfloat16), b.astype(jnp.bfloat16),
                       preferred_element_type=jnp.float32)
    g = jnp.maximum(mm(g_ref[...], w1_ref[...]) + b1_ref[...], 0.0)
    g = jnp.maximum(mm(g, w2_ref[...]) + b2_ref[...], 0.0)
    out_ref[...] = mm(g, w3_ref[...]) + b3_ref[...]


def _pad2(a, r, c):
    return jnp.pad(a, ((0, r - a.shape[0]), (0, c - a.shape[1])))


FT = T * FIN
FTP = 384
NW = 16
KB = 128
NBLK = N // NW
NTILES = 32
NCH = FTP // 16
WROWS = NW + 2
FINF = 3.0e38


def _seg_kernel(bt_hbm, ct_hbm, src_hbm, dst_hbm, attr_hbm, rp_hbm,
                s_hbm, q_hbm, mn_hbm, mx_hbm,
                ct_v, rp_v, idx_v, dstb_v, attrb_v, rows_v,
                s_w, q_w, mn_w, mx_w, sem):
    cid = lax.axis_index("c")
    sid = lax.axis_index("s")
    wid = sid * 2 + cid

    pltpu.sync_copy(ct_hbm, ct_v)
    pltpu.sync_copy(rp_hbm, rp_v)

    nk = (NBLK + NTILES - 1) // NTILES

    def block_body(k, _):
        bi = wid + NTILES * k

        @pl.when(bi < NBLK)
        def _():
            rpv = rp_v[pl.ds(bi, 16)]
            e0 = rpv[0]
            e1 = rpv[1]
            a0 = (e0 // 8) * 8
            nsub = lax.div(e1 - a0 + KB - 1, KB)

            def init_body(r, _):
                z = jnp.zeros((16,), jnp.float32)
                s_w[pl.ds(r * 16, 16)] = z
                q_w[pl.ds(r * 16, 16)] = z
                mn_w[pl.ds(r * 16, 16)] = z + FINF
                mx_w[pl.ds(r * 16, 16)] = z - FINF
                return 0
            lax.fori_loop(0, WROWS * NCH, init_body, 0)

            def sub_body(sb, _):
                estart = a0 + sb * KB
                pltpu.sync_copy(src_hbm.at[pl.ds(estart, KB)], idx_v)
                pltpu.sync_copy(dst_hbm.at[pl.ds(estart, KB)],
                                dstb_v.at[pl.ds(0, KB)])
                pltpu.sync_copy(attr_hbm.at[pl.ds(estart, KB)],
                                attrb_v.at[pl.ds(0, KB)])
                pltpu.async_copy(bt_hbm.at[idx_v], rows_v, sem).wait()

                def edge_body(e, _):
                    d = dstb_v[pl.ds(e, 16)][0]
                    a = attrb_v[pl.ds(e, 16)][0]
                    ld = jnp.minimum(jnp.maximum(d - NW * bi, -1), NW) + 1
                    off = ld * FTP
                    for j in range(NCH):
                        b = rows_v[e, pl.ds(j * 16, 16)]
                        cc = ct_v[a, pl.ds(j * 16, 16)]
                        v = b + cc
                        plsc.addupdate(s_w.at[pl.ds(off + j * 16, 16)], v)
                        plsc.addupdate(q_w.at[pl.ds(off + j * 16, 16)], v * v)
                        mslc = pl.ds(off + j * 16, 16)
                        mn_w[mslc] = jnp.minimum(mn_w[mslc], v)
                        mx_w[mslc] = jnp.maximum(mx_w[mslc], v)
                    return 0
                lax.fori_loop(0, KB, edge_body, 0)
                return 0
            lax.fori_loop(0, nsub, sub_body, 0)

            nbase = NW * bi * FTP
            pltpu.sync_copy(s_w.at[pl.ds(FTP, NW * FTP)],
                            s_hbm.at[pl.ds(nbase, NW * FTP)])
            pltpu.sync_copy(q_w.at[pl.ds(FTP, NW * FTP)],
                            q_hbm.at[pl.ds(nbase, NW * FTP)])
            pltpu.sync_copy(mn_w.at[pl.ds(FTP, NW * FTP)],
                            mn_hbm.at[pl.ds(nbase, NW * FTP)])
            pltpu.sync_copy(mx_w.at[pl.ds(FTP, NW * FTP)],
                            mx_hbm.at[pl.ds(nbase, NW * FTP)])
        return 0

    lax.fori_loop(0, nk, block_body, 0)


@jax.jit
def _seg_reduce(bt, ct, src_s, dst_s, attr_s, rp16):
    mesh = plsc.VectorSubcoreMesh(core_axis_name="c", subcore_axis_name="s")
    f32 = jnp.float32
    fn = pl.kernel(
        _seg_kernel,
        mesh=mesh,
        out_type=[jax.ShapeDtypeStruct((N * FTP,), f32) for _ in range(4)],
        scratch_types=[
            pltpu.VMEM((4, FTP), f32),
            pltpu.VMEM((RPPAD,), jnp.int32),
            pltpu.VMEM((KB,), jnp.int32),
            pltpu.VMEM((KB + 16,), jnp.int32),
            pltpu.VMEM((KB + 16,), jnp.int32),
            pltpu.VMEM((KB, FTP), f32),
            pltpu.VMEM((WROWS * FTP,), f32),
            pltpu.VMEM((WROWS * FTP,), f32),
            pltpu.VMEM((WROWS * FTP,), f32),
            pltpu.VMEM((WROWS * FTP,), f32),
            pltpu.SemaphoreType.DMA,
        ],
    )
    s4, q4, mn4, mx4 = fn(bt, ct, src_s, dst_s, attr_s, rp16)
    return (s4.reshape(N, FTP), q4.reshape(N, FTP),
            mn4.reshape(N, FTP), mx4.reshape(N, FTP))


EPAD = E + 2 * KB
RPPAD = 648


def _bf(a):
    return a.astype(jnp.bfloat16)


def _mm(a, b):
    return jnp.dot(_bf(a), _bf(b), preferred_element_type=jnp.float32)


def _es(spec, a, b):
    return jnp.einsum(spec, _bf(a), _bf(b), preferred_element_type=jnp.float32)


def kernel(*args):
    return _impl(*args)


def _impl(x, edge_index, edge_attr, batch, node_emb, edge_emb, edge_enc_W, edge_enc_b, pre_W, pre_b, post_W, post_b, lin_W, lin_b, bn_gamma, bn_beta, mlp_W1, mlp_b1, mlp_W2, mlp_b2, mlp_W3, mlp_b3):
    h = node_emb[x]
    src, dst = edge_index[0], edge_index[1]
    perm = jnp.argsort(dst)
    src_s = src[perm].astype(jnp.int32)
    dst_s = dst[perm].astype(jnp.int32)
    attr_s = edge_attr[perm].astype(jnp.int32)
    rp16 = jnp.searchsorted(dst_s, jnp.arange(NBLK + 1) * NW).astype(jnp.int32)
    rp16 = jnp.pad(rp16, (0, RPPAD - NBLK - 1), constant_values=E)
    src_p = jnp.pad(src_s, (0, EPAD - E))
    dst_p = jnp.pad(dst_s, (0, EPAD - E), constant_values=N)
    attr_p = jnp.pad(attr_s, (0, EPAD - E))
    cnt_c = None
    empty = None
    TF = T * FIN
    for l in range(L):
        ee4 = _mm(edge_emb, edge_enc_W[l]) + edge_enc_b[l]
        A = (_es('nf,tfo->nto', h, pre_W[l][:, :FIN]) + pre_b[l]).reshape(N, TF)
        B = _es('nf,tfo->nto', h, pre_W[l][:, FIN:2 * FIN]).reshape(N, TF)
        C = _es('kf,tfo->kto', ee4, pre_W[l][:, 2 * FIN:]).reshape(4, TF)
        bt = jnp.concatenate(
            [B, jnp.ones((N, 1), jnp.float32),
             jnp.zeros((N, FTP - FT - 1), jnp.float32)], axis=1)
        ct = jnp.pad(C, ((0, 0), (0, FTP - FT)))
        S4, Q4, MN4, MX4 = _seg_reduce(bt, ct, src_p, dst_p, attr_p, rp16)
        if l == 0:
            cnt = S4[:, FT]
            cnt_c = jnp.maximum(cnt, 1.0)[:, None]
            empty = (cnt == 0.0)[:, None]
        S = S4[:, :TF]
        Q = Q4[:, :TF]
        MN = MN4[:, :TF]
        MX = MX4[:, :TF]
        meanv = S / cnt_c
        mean = jnp.where(empty, 0.0, A + meanv)
        std = jnp.sqrt(jax.nn.relu(Q / cnt_c - meanv * meanv) + 1e-5)
        mn = jnp.where(empty, 0.0, A + MN)
        mx = jnp.where(empty, 0.0, A + MX)
        P = post_W[l]
        out = (_es('nf,tfo->nto', h, P[:, :FIN])
               + _es('ntf,tfo->nto', mean.reshape(N, T, FIN), P[:, FIN:2 * FIN])
               + _es('ntf,tfo->nto', mn.reshape(N, T, FIN), P[:, 2 * FIN:3 * FIN])
               + _es('ntf,tfo->nto', mx.reshape(N, T, FIN), P[:, 3 * FIN:4 * FIN])
               + _es('ntf,tfo->nto', std.reshape(N, T, FIN), P[:, 4 * FIN:])
               + post_b[l])
        out = _mm(out.reshape(N, T * FOUT), lin_W[l]) + lin_b[l]
        mu = out.mean(axis=0)
        var = out.var(axis=0)
        out = (out - mu) / jnp.sqrt(var + 1e-5) * bn_gamma[l] + bn_beta[l]
        h = jax.nn.relu(out)

    hp = _pad2(h, N, FP)
    g = pl.pallas_call(
        _pool_kernel,
        grid=(N // BLK,),
        in_specs=[pl.BlockSpec((1, 1, BLK), lambda i: (i, 0, 0)),
                  pl.BlockSpec((BLK, FP), lambda i: (i, 0))],
        out_specs=pl.BlockSpec((G, FP), lambda i: (0, 0)),
        out_shape=jax.ShapeDtypeStruct((G, FP), jnp.float32),
        scratch_shapes=[pltpu.VMEM((G, FP), jnp.float32)],
    )(batch.astype(jnp.int32).reshape(N // BLK, 1, BLK), hp)

    w1 = _pad2(mlp_W1, FP, FP)
    b1 = _pad2(mlp_b1.reshape(1, -1), 1, FP)
    w2 = _pad2(mlp_W2, FP, FP)
    b2 = _pad2(mlp_b2.reshape(1, -1), 1, FP)
    w3 = _pad2(mlp_W3, FP, FP)
    b3 = _pad2(mlp_b3.reshape(1, -1), 1, FP)
    out = pl.pallas_call(
        _mlp_kernel,
        out_shape=jax.ShapeDtypeStruct((G, FP), jnp.float32),
    )(g, w1, b1, w2, b2, w3, b3)
    return out[:, :1]


# speedup vs baseline: 59.8238x; 1.7789x over previous
import functools

import jax
import jax.numpy as jnp
from jax import lax
from jax.experimental import pallas as pl
from jax.experimental.pallas import tpu as pltpu
from jax.experimental.pallas import tpu_sc as plsc

N = 10000
E = 160000
G = 256
L = 4
T = 5
FIN = 75
FOUT = 15
FP = 128
BLK = 1000


def _pool_kernel(batch_ref, raw_ref, st_ref, gam_ref, bet_ref, out_ref, acc):
    i = pl.program_id(0)
    nblk = pl.num_programs(0)
    b = batch_ref[0]
    hb = _bn_relu(raw_ref[...], st_ref[...], gam_ref[...][0:1],
                  bet_ref[...][0:1])
    gids = lax.broadcasted_iota(jnp.int32, (G, BLK), 0)
    onehot = (gids == b).astype(jnp.float32)
    contrib = jnp.dot(onehot, hb, preferred_element_type=jnp.float32,
                      precision=lax.Precision.HIGHEST)

    @pl.when(i == 0)
    def _():
        acc[...] = jnp.zeros_like(acc)

    acc[...] += contrib

    @pl.when(i == nblk - 1)
    def _():
        out_ref[...] = acc[...]


def _mlp_kernel(g_ref, w1_ref, b1_ref, w2_ref, b2_ref, w3_ref, b3_ref, out_ref):
    def mm(a, b):
        return jnp.dot(a.astype(jnp.bfloat16), b.astype(jnp.bfloat16),
                       preferred_element_type=jnp.float32)
    g = jnp.maximum(mm(g_ref[...], w1_ref[...]) + b1_ref[...], 0.0)
    g = jnp.maximum(mm(g, w2_ref[...]) + b2_ref[...], 0.0)
    out_ref[...] = mm(g, w3_ref[...]) + b3_ref[...]


def _pad2(a, r, c):
    return jnp.pad(a, ((0, r - a.shape[0]), (0, c - a.shape[1])))


def _dot16(a, b):
    return jnp.dot(a.astype(jnp.bfloat16), b.astype(jnp.bfloat16),
                   preferred_element_type=jnp.float32)


def _bn_relu(raw, st, gamma, beta):
    mu = st[0:1] / N
    var = st[1:2] / N
    return jnp.maximum((raw - mu) / jnp.sqrt(var + 1e-5) * gamma + beta, 0.0)


def _var_kernel(raw_ref, st_ref, st2_ref, acc):
    i = pl.program_id(0)
    nblk = pl.num_programs(0)
    mu = st_ref[...][0:1] / N
    d = raw_ref[...] - mu
    ssd = jnp.sum(d * d, 0, keepdims=True)

    @pl.when(i == 0)
    def _():
        acc[...] = jnp.zeros_like(acc)

    acc[...] += jnp.concatenate(
        [jnp.zeros((1, FP), jnp.float32), ssd,
         jnp.zeros((6, FP), jnp.float32)], axis=0)

    @pl.when(i == nblk - 1)
    def _():
        st2_ref[...] = st_ref[...] * jnp.where(
            lax.broadcasted_iota(jnp.int32, (8, FP), 0) == 0, 1.0, 0.0) + acc[...]


DBLK = 1000


def _k1bn_kernel(raw_ref, st_ref, gam_ref, bet_ref, wd_ref, ws_ref,
                 preb_ref, ctp_ref, h_ref, a_ref, b4_ref):
    h = _bn_relu(raw_ref[...], st_ref[...], gam_ref[...][0:1],
                 bet_ref[...][0:1])
    h_ref[...] = h
    a_ref[...] = _dot16(h, wd_ref[...]) + preb_ref[...][0:1]
    b = _dot16(h, ws_ref[...])
    b4_ref[...] = b[:, None, :] + ctp_ref[...][0:4][None, :, :]


def _k0_kernel(x_ref, emb_ref, wd_ref, ws_ref, preb_ref, ctp_ref,
               h_ref, a_ref, b4_ref):
    xv = x_ref[0]
    vids = lax.broadcasted_iota(jnp.int32, (32, DBLK), 0)
    onehot = (vids == xv).astype(jnp.float32)
    h = jnp.einsum('vn,vf->nf', onehot, emb_ref[...],
                   preferred_element_type=jnp.float32,
                   precision=lax.Precision.HIGHEST)
    h_ref[...] = h
    a_ref[...] = _dot16(h, wd_ref[...]) + preb_ref[...][0:1]
    b = _dot16(h, ws_ref[...])
    b4_ref[...] = b[:, None, :] + ctp_ref[...][0:4][None, :, :]


def _k2_kernel(s_ref, q_ref, mn_ref, mx_ref, a_ref, h_ref,
               p1_ref, bd2_ref, bd3_ref, bd4_ref, bd5_ref,
               postb_ref, lin_ref, linb_ref,
               out_ref, st_ref, stacc):
    i = pl.program_id(0)
    nblk = pl.num_programs(0)
    s4 = s_ref[...]
    a4 = a_ref[...]
    cnt = s4[:, 375:376]
    cntc = jnp.maximum(cnt, 1.0)
    empty = cnt == 0.0
    meanv = s4 / cntc
    mean = jnp.where(empty, 0.0, a4 + meanv)
    std = jnp.sqrt(jnp.maximum(q_ref[...] / cntc - meanv * meanv, 0.0) + 1e-5)
    mn = jnp.where(empty, 0.0, a4 + mn_ref[...])
    mx = jnp.where(empty, 0.0, a4 + mx_ref[...])
    y = (_dot16(h_ref[...], p1_ref[...]) + _dot16(mean, bd2_ref[...])
         + _dot16(mn, bd3_ref[...]) + _dot16(mx, bd4_ref[...])
         + _dot16(std, bd5_ref[...]) + postb_ref[...][0:1])
    out = _dot16(y, lin_ref[...]) + linb_ref[...][0:1]
    out_ref[...] = out

    @pl.when(i == 0)
    def _():
        stacc[...] = jnp.zeros_like(stacc)

    z = jnp.zeros((6, FP), jnp.float32)
    stacc[...] += jnp.concatenate(
        [jnp.sum(out, 0, keepdims=True), jnp.sum(out * out, 0, keepdims=True),
         z], axis=0)

    @pl.when(i == nblk - 1)
    def _():
        st_ref[...] = stacc[...]


FT = T * FIN
FTP = 384
NW = 16
KB = 128
NBLK = N // NW
NTILES = 32
NCH = FTP // 16
WROWS = NW + 2
FINF = 3.0e38


NPASS = 2
CPP = NCH // NPASS


def _seg_kernel(bt_hbm, src_hbm, dst_hbm, rp_hbm,
                s_hbm, q_hbm, mn_hbm, mx_hbm,
                rp_v, idx_v, dstb_v, rows_v,
                s_w, q_w, mn_w, mx_w, sem):
    cid = lax.axis_index("c")
    sid = lax.axis_index("s")
    wid = sid * 2 + cid

    pltpu.sync_copy(rp_hbm, rp_v)

    nk = (NBLK + NTILES - 1) // NTILES
    zv = jnp.zeros((16,), jnp.float32)
    ident = ([zv] * CPP) + ([zv] * CPP) + ([zv + FINF] * CPP) + ([zv - FINF] * CPP)

    def combine(row, accs, cbase):
        for j in range(CPP):
            off = row * FTP + cbase + j * 16
            slc = pl.ds(off, 16)
            plsc.addupdate(s_w.at[slc], accs[j])
            plsc.addupdate(q_w.at[slc], accs[CPP + j])
            mn_w[slc] = jnp.minimum(mn_w[slc], accs[2 * CPP + j])
            mx_w[slc] = jnp.maximum(mx_w[slc], accs[3 * CPP + j])

    def block_body(k, _):
        bi = wid + NTILES * k

        @pl.when(bi < NBLK)
        def _():
            rpv = rp_v[pl.ds(bi, 16)]
            e0 = rpv[0]
            e1 = rpv[1]
            a0 = (e0 // 8) * 8
            nsub = lax.div(e1 - a0 + KB - 1, KB)

            def init_body(r, _):
                s_w[pl.ds(r * 16, 16)] = zv
                q_w[pl.ds(r * 16, 16)] = zv
                mn_w[pl.ds(r * 16, 16)] = zv + FINF
                mx_w[pl.ds(r * 16, 16)] = zv - FINF
                return 0
            lax.fori_loop(0, WROWS * NCH, init_body, 0)

            def sub_body(sb, _):
                estart = a0 + sb * KB
                pltpu.sync_copy(src_hbm.at[pl.ds(estart, KB)], idx_v)
                pltpu.sync_copy(dst_hbm.at[pl.ds(estart, KB)],
                                dstb_v.at[pl.ds(0, KB)])
                pltpu.async_copy(bt_hbm.at[idx_v], rows_v, sem).wait()

                for p in range(NPASS):
                    cbase = p * CPP * 16

                    def edge_body(e, carry):
                        row = carry[0]
                        accs = list(carry[1:])
                        d = dstb_v[pl.ds(e, 16)][0]
                        nrow = jnp.minimum(jnp.maximum(d - NW * bi, -1), NW) + 1

                        def do_flush(op):
                            prow, paccs = op[0], list(op[1:])
                            combine(prow, paccs, cbase)
                            return tuple([prow] + ident)

                        def no_flush(op):
                            return tuple(op)

                        res = lax.cond(nrow != row, do_flush, no_flush,
                                       tuple([row] + accs))
                        accs = list(res[1:])
                        out = []
                        for j in range(CPP):
                            b = rows_v[e, pl.ds(cbase + j * 16, 16)]
                            out.append(accs[j] + b)
                        for j in range(CPP):
                            b = rows_v[e, pl.ds(cbase + j * 16, 16)]
                            out.append(accs[CPP + j] + b * b)
                        for j in range(CPP):
                            b = rows_v[e, pl.ds(cbase + j * 16, 16)]
                            out.append(jnp.minimum(accs[2 * CPP + j], b))
                        for j in range(CPP):
                            b = rows_v[e, pl.ds(cbase + j * 16, 16)]
                            out.append(jnp.maximum(accs[3 * CPP + j], b))
                        return tuple([nrow] + out)

                    fin = lax.fori_loop(0, KB, edge_body, tuple([0] + ident))
                    combine(fin[0], list(fin[1:]), cbase)
                return 0
            lax.fori_loop(0, nsub, sub_body, 0)

            nbase = NW * bi * FTP
            pltpu.sync_copy(s_w.at[pl.ds(FTP, NW * FTP)],
                            s_hbm.at[pl.ds(nbase, NW * FTP)])
            pltpu.sync_copy(q_w.at[pl.ds(FTP, NW * FTP)],
                            q_hbm.at[pl.ds(nbase, NW * FTP)])
            pltpu.sync_copy(mn_w.at[pl.ds(FTP, NW * FTP)],
                            mn_hbm.at[pl.ds(nbase, NW * FTP)])
            pltpu.sync_copy(mx_w.at[pl.ds(FTP, NW * FTP)],
                            mx_hbm.at[pl.ds(nbase, NW * FTP)])
        return 0

    lax.fori_loop(0, nk, block_body, 0)


@jax.jit
def _seg_reduce(bt4, src4, dst_s, rp16):
    mesh = plsc.VectorSubcoreMesh(core_axis_name="c", subcore_axis_name="s")
    f32 = jnp.float32
    fn = pl.kernel(
        _seg_kernel,
        mesh=mesh,
        out_type=[jax.ShapeDtypeStruct((N * FTP,), f32) for _ in range(4)],
        scratch_types=[
            pltpu.VMEM((RPPAD,), jnp.int32),
            pltpu.VMEM((KB,), jnp.int32),
            pltpu.VMEM((KB + 16,), jnp.int32),
            pltpu.VMEM((KB, FTP), f32),
            pltpu.VMEM((WROWS * FTP,), f32),
            pltpu.VMEM((WROWS * FTP,), f32),
            pltpu.VMEM((WROWS * FTP,), f32),
            pltpu.VMEM((WROWS * FTP,), f32),
            pltpu.SemaphoreType.DMA,
        ],
    )
    s4, q4, mn4, mx4 = fn(bt4, src4, dst_s, rp16)
    return (s4.reshape(N, FTP), q4.reshape(N, FTP),
            mn4.reshape(N, FTP), mx4.reshape(N, FTP))


EPAD = E + 2 * KB
RPPAD = 648


def _bf(a):
    return a.astype(jnp.bfloat16)


def _mm(a, b):
    return jnp.dot(_bf(a), _bf(b), preferred_element_type=jnp.float32)


def _es(spec, a, b):
    return jnp.einsum(spec, _bf(a), _bf(b), preferred_element_type=jnp.float32)


def kernel(*args):
    return _impl(*args)


def _impl(x, edge_index, edge_attr, batch, node_emb, edge_emb, edge_enc_W, edge_enc_b, pre_W, pre_b, post_W, post_b, lin_W, lin_b, bn_gamma, bn_beta, mlp_W1, mlp_b1, mlp_W2, mlp_b2, mlp_W3, mlp_b3):
    src, dst = edge_index[0], edge_index[1]
    perm = jnp.argsort(dst)
    src_s = src[perm].astype(jnp.int32)
    dst_s = dst[perm].astype(jnp.int32)
    attr_s = edge_attr[perm].astype(jnp.int32)
    rp16 = jnp.searchsorted(dst_s, jnp.arange(NBLK + 1) * NW).astype(jnp.int32)
    rp16 = jnp.pad(rp16, (0, RPPAD - NBLK - 1), constant_values=E)
    src4_p = jnp.pad(4 * src_s + attr_s, (0, EPAD - E))
    dst_p = jnp.pad(dst_s, (0, EPAD - E), constant_values=N)
    TF = T * FIN
    NB = N // DBLK
    full2 = lambda shape: pl.BlockSpec(shape, lambda i: (0, 0))
    blk2 = lambda c: pl.BlockSpec((DBLK, c), lambda i: (i, 0))
    x3 = x.astype(jnp.int32).reshape(NB, 1, DBLK)
    embp = _pad2(node_emb, 32, FP)
    f32 = jnp.float32
    raw = st = gam = bet = None
    for l in range(L):
        ee4 = _mm(edge_emb, edge_enc_W[l]) + edge_enc_b[l]
        C = _es('kf,tfo->kto', ee4, pre_W[l][:, 2 * FIN:]).reshape(4, TF)
        ctp = jnp.pad(C, ((0, 4), (0, FTP - FT)))
        ctp = ctp.at[0:4, FT].set(1.0)
        wd = _pad2(pre_W[l][:, :FIN].transpose(1, 0, 2).reshape(FIN, TF),
                   FP, FTP)
        ws = _pad2(pre_W[l][:, FIN:2 * FIN].transpose(1, 0, 2).reshape(FIN, TF),
                   FP, FTP)
        preb = _pad2(pre_b[l].reshape(1, TF), 8, FTP)
        wspecs = [full2((FP, FTP)), full2((FP, FTP)), full2((8, FTP)),
                  full2((8, FTP))]
        outspecs = [blk2(FP), blk2(FTP),
                    pl.BlockSpec((DBLK, 4, FTP), lambda i: (i, 0, 0))]
        outshapes = [jax.ShapeDtypeStruct((N, FP), f32),
                     jax.ShapeDtypeStruct((N, FTP), f32),
                     jax.ShapeDtypeStruct((N, 4, FTP), f32)]
        if l == 0:
            h, A, b4 = pl.pallas_call(
                _k0_kernel, grid=(NB,),
                in_specs=[pl.BlockSpec((1, 1, DBLK), lambda i: (i, 0, 0)),
                          full2((32, FP))] + wspecs,
                out_specs=outspecs, out_shape=outshapes,
            )(x3, embp, wd, ws, preb, ctp)
        else:
            h, A, b4 = pl.pallas_call(
                _k1bn_kernel, grid=(NB,),
                in_specs=[blk2(FP), full2((8, FP)), full2((8, FP)),
                          full2((8, FP))] + wspecs,
                out_specs=outspecs, out_shape=outshapes,
            )(raw, st, gam, bet, wd, ws, preb, ctp)

        S4, Q4, MN4, MX4 = _seg_reduce(b4.reshape(4 * N, FTP),
                                       src4_p, dst_p, rp16)

        P = post_W[l]
        eye = jnp.eye(T, dtype=f32)
        p1 = _pad2(P[:, :FIN].transpose(1, 0, 2).reshape(FIN, T * FOUT),
                   FP, FP)
        bds = [_pad2(jnp.einsum('tfo,ts->tfso', P[:, c * FIN:(c + 1) * FIN],
                                eye).reshape(TF, T * FOUT), FTP, FP)
               for c in range(1, 5)]
        postb = _pad2(post_b[l].reshape(1, T * FOUT), 8, FP)
        linp = _pad2(lin_W[l], FP, FP)
        linb = _pad2(lin_b[l].reshape(1, -1), 8, FP)
        gam = _pad2(bn_gamma[l].reshape(1, -1), 8, FP)
        bet = _pad2(bn_beta[l].reshape(1, -1), 8, FP)
        raw, st = pl.pallas_call(
            _k2_kernel, grid=(NB,),
            in_specs=[blk2(FTP)] * 5 + [blk2(FP)]
            + [full2((FP, FP))] + [full2((FTP, FP))] * 4
            + [full2((8, FP)), full2((FP, FP)), full2((8, FP))],
            out_specs=[blk2(FP), pl.BlockSpec((8, FP), lambda i: (0, 0))],
            out_shape=[jax.ShapeDtypeStruct((N, FP), f32),
                       jax.ShapeDtypeStruct((8, FP), f32)],
            scratch_shapes=[pltpu.VMEM((8, FP), f32)],
        )(S4, Q4, MN4, MX4, A, h, p1, *bds, postb, linp, linb)
        st = pl.pallas_call(
            _var_kernel, grid=(NB,),
            in_specs=[blk2(FP), pl.BlockSpec((8, FP), lambda i: (0, 0))],
            out_specs=pl.BlockSpec((8, FP), lambda i: (0, 0)),
            out_shape=jax.ShapeDtypeStruct((8, FP), f32),
            scratch_shapes=[pltpu.VMEM((8, FP), f32)],
        )(raw, st)

    g = pl.pallas_call(
        _pool_kernel,
        grid=(N // BLK,),
        in_specs=[pl.BlockSpec((1, 1, BLK), lambda i: (i, 0, 0)),
                  pl.BlockSpec((BLK, FP), lambda i: (i, 0)),
                  pl.BlockSpec((8, FP), lambda i: (0, 0)),
                  pl.BlockSpec((8, FP), lambda i: (0, 0)),
                  pl.BlockSpec((8, FP), lambda i: (0, 0))],
        out_specs=pl.BlockSpec((G, FP), lambda i: (0, 0)),
        out_shape=jax.ShapeDtypeStruct((G, FP), jnp.float32),
        scratch_shapes=[pltpu.VMEM((G, FP), jnp.float32)],
    )(batch.astype(jnp.int32).reshape(N // BLK, 1, BLK), raw, st, gam, bet)

    w1 = _pad2(mlp_W1, FP, FP)
    b1 = _pad2(mlp_b1.reshape(1, -1), 1, FP)
    w2 = _pad2(mlp_W2, FP, FP)
    b2 = _pad2(mlp_b2.reshape(1, -1), 1, FP)
    w3 = _pad2(mlp_W3, FP, FP)
    b3 = _pad2(mlp_b3.reshape(1, -1), 1, FP)
    out = pl.pallas_call(
        _mlp_kernel,
        out_shape=jax.ShapeDtypeStruct((G, FP), jnp.float32),
    )(g, w1, b1, w2, b2, w3, b3)
    return out[:, :1]
